# Initial kernel scaffold; baseline (speedup 1.0000x reference)
import jax
import jax.numpy as jnp
from jax.experimental import pallas as pl


def kernel(boil_time, mash_ph, sparge_temp, mash_step_type_inds, mash_step_times, mash_step_avg_temps, ferment_stage_times, ferment_stage_temps, grain_core_type_inds, grain_amts, adjunct_core_type_inds, adjunct_amts, hop_type_inds, hop_stage_type_inds, hop_times, hop_concentrations, misc_type_inds, misc_stage_inds, misc_times, misc_amts, mo_type_inds, mo_stage_inds, W_mash_step_type, W_grain_type, W_adjunct_type, W_hop_type, W_hop_stage_type, W_misc_type, W_misc_stage_type, W_mo_type, W_mo_stage_type):
    raise NotImplementedError("write your pallas kernel here")


# speedup vs baseline: 1.8144x; 1.8144x over previous
import functools

import jax
import jax.numpy as jnp
from jax import lax
from jax.experimental import pallas as pl
from jax.experimental.pallas import tpu as pltpu
from jax.experimental.pallas import tpu_sc as plsc

B = 4096
NC, NS = 2, 16
NW = NC * NS
BPW = B // NW
ROWS_C = 8
NCHUNK = BPW // ROWS_C
NCOLS = 4419


def _pad16(d):
    return max(16, (d + 15) // 16 * 16)


EMB = [
    dict(name="mash",       S=6,  D=4,  c0=3),
    dict(name="grain",      S=16, D=32, c0=43),
    dict(name="adjunct",    S=8,  D=32, c0=571),
    dict(name="hop",        S=32, D=64, c0=835),
    dict(name="hop_stage",  S=32, D=8,  c0=2883),
    dict(name="misc",       S=16, D=32, c0=3203),
    dict(name="misc_stage", S=16, D=8,  c0=3715),
    dict(name="mo",         S=8,  D=64, c0=3875),
    dict(name="mo_stage",   S=8,  D=4,  c0=4387),
]
for f in EMB:
    f["Dp"] = _pad16(f["D"])
    f["nidx"] = ROWS_C * f["S"]
    f["npad"] = (f["nidx"] + 127) // 128 * 128

PT = [
    dict(name="boil_time",           S=1,  c0=0),
    dict(name="mash_ph",             S=1,  c0=1),
    dict(name="sparge_temp",         S=1,  c0=2),
    dict(name="mash_step_times",     S=6,  c0=27),
    dict(name="mash_step_avg_temps", S=6,  c0=33),
    dict(name="ferment_stage_times", S=2,  c0=39),
    dict(name="ferment_stage_temps", S=2,  c0=41),
    dict(name="grain_amts",          S=16, c0=555),
    dict(name="adjunct_amts",        S=8,  c0=827),
    dict(name="hop_times",           S=32, c0=3139),
    dict(name="hop_concentrations",  S=32, c0=3171),
    dict(name="misc_times",          S=16, c0=3843),
    dict(name="misc_amts",           S=16, c0=3859),
]

SWEEP = sorted(
    [("emb", i, f["c0"]) for i, f in enumerate(EMB)]
    + [("pt", i, f["c0"]) for i, f in enumerate(PT)],
    key=lambda t: t[2])


def _arrange(inds):
    S = inds.shape[1]
    nidx = ROWS_C * S
    npad = (nidx + 127) // 128 * 128
    a = inds.astype(jnp.int32).reshape(NW * NCHUNK, nidx)
    if npad != nidx:
        a = jnp.pad(a, ((0, 0), (0, npad - nidx)))
    return a


def _body(*refs):
    n_emb, n_pt = len(EMB), len(PT)
    idx_refs = refs[:n_emb]
    tbl_refs = refs[n_emb:2 * n_emb]
    pt_refs = refs[2 * n_emb:2 * n_emb + n_pt]
    out = refs[2 * n_emb + n_pt]
    rest = refs[2 * n_emb + n_pt + 1:]
    idx_bufs = rest[:n_emb]
    gbufs = rest[n_emb:2 * n_emb]
    pt_bufs = rest[2 * n_emb:2 * n_emb + n_pt]
    rowbuf = rest[2 * n_emb + n_pt]
    sem = rest[2 * n_emb + n_pt + 1]

    wid = lax.axis_index("s") * NC + lax.axis_index("c")
    r0 = wid * BPW

    for f, ref, buf in zip(PT, pt_refs, pt_bufs):
        S = f["S"]
        pltpu.sync_copy(ref.at[pl.ds(r0 * S, BPW * S)],
                        buf.at[pl.ds(0, BPW * S)])

    def chunk_body(c, _):
        for f, iref, ibuf, gbuf, tref in zip(EMB, idx_refs, idx_bufs, gbufs,
                                             tbl_refs):
            pltpu.sync_copy(iref.at[wid * NCHUNK + c], ibuf)
            for j in range(f["npad"] // 128):
                pltpu.async_copy(
                    tref.at[ibuf.at[pl.ds(j * 128, 128)]],
                    gbuf.at[pl.ds(j * 128, 128), :], sem).wait()

        def row_body(r, _):
            rb = r * NCOLS
            for kind, i, c0 in SWEEP:
                if kind == "emb":
                    f, gbuf = EMB[i], gbufs[i]
                    S, D, Dp = f["S"], f["D"], f["Dp"]
                    for s in range(S):
                        n = r * S + s
                        for j in range(Dp // 16):
                            rowbuf[pl.ds(rb + c0 + s * D + j * 16, 16)] = (
                                gbuf[n, pl.ds(j * 16, 16)])
                else:
                    f, buf = PT[i], pt_bufs[i]
                    S = f["S"]
                    base = (c * ROWS_C + r) * S
                    for k in range((S + 15) // 16):
                        rowbuf[pl.ds(rb + c0 + k * 16, 16)] = (
                            buf[pl.ds(base + k * 16, 16)])
            return _

        lax.fori_loop(0, ROWS_C, row_body, None)

        pltpu.sync_copy(
            rowbuf.at[pl.ds(0, ROWS_C * NCOLS)],
            out.at[pl.ds((r0 + c * ROWS_C) * NCOLS, ROWS_C * NCOLS)])
        return _

    lax.fori_loop(0, NCHUNK, chunk_body, None)


@jax.jit
def _run(idx_args, tbl_args, pt_args):
    mesh = plsc.VectorSubcoreMesh(core_axis_name="c", subcore_axis_name="s")
    scratch = [pltpu.VMEM((f["npad"],), jnp.int32) for f in EMB]
    scratch += [pltpu.VMEM((f["npad"], f["Dp"]), jnp.float32) for f in EMB]
    scratch += [pltpu.VMEM((BPW * f["S"] + 16,), jnp.float32) for f in PT]
    scratch += [pltpu.VMEM((ROWS_C * NCOLS + 16,), jnp.float32),
                pltpu.SemaphoreType.DMA]
    k = pl.kernel(
        _body,
        mesh=mesh,
        out_type=jax.ShapeDtypeStruct((B * NCOLS,), jnp.float32),
        scratch_types=scratch,
        compiler_params=pltpu.CompilerParams(use_tc_tiling_on_sc=False),
    )
    return k(*idx_args, *tbl_args, *pt_args).reshape(B, NCOLS)


def kernel(boil_time, mash_ph, sparge_temp, mash_step_type_inds,
           mash_step_times, mash_step_avg_temps, ferment_stage_times,
           ferment_stage_temps, grain_core_type_inds, grain_amts,
           adjunct_core_type_inds, adjunct_amts, hop_type_inds,
           hop_stage_type_inds, hop_times, hop_concentrations,
           misc_type_inds, misc_stage_inds, misc_times, misc_amts,
           mo_type_inds, mo_stage_inds, W_mash_step_type, W_grain_type,
           W_adjunct_type, W_hop_type, W_hop_stage_type, W_misc_type,
           W_misc_stage_type, W_mo_type, W_mo_stage_type):
    idx_inputs = [mash_step_type_inds, grain_core_type_inds,
                  adjunct_core_type_inds, hop_type_inds, hop_stage_type_inds,
                  misc_type_inds, misc_stage_inds, mo_type_inds,
                  mo_stage_inds]
    idx_args = [_arrange(a) for a in idx_inputs]
    tbls = [W_mash_step_type, W_grain_type, W_adjunct_type, W_hop_type,
            W_hop_stage_type, W_misc_type, W_misc_stage_type, W_mo_type,
            W_mo_stage_type]
    tbl_args = [
        jnp.pad(t, ((0, 0), (0, f["Dp"] - f["D"]))) if f["Dp"] != f["D"] else t
        for t, f in zip(tbls, EMB)]
    pt_args = [boil_time, mash_ph, sparge_temp,
               mash_step_times.reshape(-1), mash_step_avg_temps.reshape(-1),
               ferment_stage_times.reshape(-1), ferment_stage_temps.reshape(-1),
               grain_amts.reshape(-1), adjunct_amts.reshape(-1),
               hop_times.reshape(-1), hop_concentrations.reshape(-1),
               misc_times.reshape(-1), misc_amts.reshape(-1)]
    return _run(idx_args, tbl_args, pt_args)


# speedup vs baseline: 1.8715x; 1.0315x over previous
import functools

import jax
import jax.numpy as jnp
from jax import lax
from jax.experimental import pallas as pl
from jax.experimental.pallas import tpu as pltpu
from jax.experimental.pallas import tpu_sc as plsc

B = 4096
NC, NS = 2, 16
NW = NC * NS
BPW = B // NW
ROWS_C = 8
NCHUNK = BPW // ROWS_C
NCOLS = 4419


def _pad16(d):
    return max(16, (d + 15) // 16 * 16)


EMB = [
    dict(name="mash",       S=6,  D=4,  c0=3),
    dict(name="grain",      S=16, D=32, c0=43),
    dict(name="adjunct",    S=8,  D=32, c0=571),
    dict(name="hop",        S=32, D=64, c0=835),
    dict(name="hop_stage",  S=32, D=8,  c0=2883),
    dict(name="misc",       S=16, D=32, c0=3203),
    dict(name="misc_stage", S=16, D=8,  c0=3715),
    dict(name="mo",         S=8,  D=64, c0=3875),
    dict(name="mo_stage",   S=8,  D=4,  c0=4387),
]
for f in EMB:
    f["Dp"] = _pad16(f["D"])
    f["nidx"] = ROWS_C * f["S"]
    f["npad"] = (f["nidx"] + 127) // 128 * 128

PT = [
    dict(name="boil_time",           S=1,  c0=0),
    dict(name="mash_ph",             S=1,  c0=1),
    dict(name="sparge_temp",         S=1,  c0=2),
    dict(name="mash_step_times",     S=6,  c0=27),
    dict(name="mash_step_avg_temps", S=6,  c0=33),
    dict(name="ferment_stage_times", S=2,  c0=39),
    dict(name="ferment_stage_temps", S=2,  c0=41),
    dict(name="grain_amts",          S=16, c0=555),
    dict(name="adjunct_amts",        S=8,  c0=827),
    dict(name="hop_times",           S=32, c0=3139),
    dict(name="hop_concentrations",  S=32, c0=3171),
    dict(name="misc_times",          S=16, c0=3843),
    dict(name="misc_amts",           S=16, c0=3859),
]

SWEEP = sorted(
    [("emb", i, f["c0"]) for i, f in enumerate(EMB)]
    + [("pt", i, f["c0"]) for i, f in enumerate(PT)],
    key=lambda t: t[2])


def _arrange(inds):
    S = inds.shape[1]
    nidx = ROWS_C * S
    npad = (nidx + 127) // 128 * 128
    a = inds.astype(jnp.int32).reshape(NW * NCHUNK, nidx)
    if npad != nidx:
        a = jnp.pad(a, ((0, 0), (0, npad - nidx)))
    return a


def _body(*refs):
    n_emb, n_pt = len(EMB), len(PT)
    idx_refs = refs[:n_emb]
    tbl_refs = refs[n_emb:2 * n_emb]
    pt_refs = refs[2 * n_emb:2 * n_emb + n_pt]
    out = refs[2 * n_emb + n_pt]
    rest = refs[2 * n_emb + n_pt + 1:]
    idx_bufs = rest[:n_emb]
    gbufs = rest[n_emb:2 * n_emb]
    pt_bufs = rest[2 * n_emb:2 * n_emb + n_pt]
    rowbuf = rest[2 * n_emb + n_pt]
    sem = rest[2 * n_emb + n_pt + 1]

    wid = lax.axis_index("s") * NC + lax.axis_index("c")
    r0 = wid * BPW

    pt_copies = [
        pltpu.async_copy(ref.at[pl.ds(r0 * f["S"], BPW * f["S"])],
                         buf.at[pl.ds(0, BPW * f["S"])], sem)
        for f, ref, buf in zip(PT, pt_refs, pt_bufs)]
    for cp in pt_copies:
        cp.wait()

    def chunk_body(c, _):
        idx_copies = [
            pltpu.async_copy(iref.at[wid * NCHUNK + c], ibuf, sem)
            for iref, ibuf in zip(idx_refs, idx_bufs)]
        for icp in idx_copies:
            icp.wait()
        gathers = []
        for f, ibuf, gbuf, tref in zip(EMB, idx_bufs, gbufs, tbl_refs):
            for j in range(f["npad"] // 128):
                gathers.append(pltpu.async_copy(
                    tref.at[ibuf.at[pl.ds(j * 128, 128)]],
                    gbuf.at[pl.ds(j * 128, 128), :], sem))
        for g in gathers:
            g.wait()

        def row_body(r, _):
            rb = r * NCOLS
            for kind, i, c0 in SWEEP:
                if kind == "emb":
                    f, gbuf = EMB[i], gbufs[i]
                    S, D, Dp = f["S"], f["D"], f["Dp"]
                    for s in range(S):
                        n = r * S + s
                        for j in range(Dp // 16):
                            rowbuf[pl.ds(rb + c0 + s * D + j * 16, 16)] = (
                                gbuf[n, pl.ds(j * 16, 16)])
                else:
                    f, buf = PT[i], pt_bufs[i]
                    S = f["S"]
                    base = (c * ROWS_C + r) * S
                    for k in range((S + 15) // 16):
                        rowbuf[pl.ds(rb + c0 + k * 16, 16)] = (
                            buf[pl.ds(base + k * 16, 16)])
            return _

        lax.fori_loop(0, ROWS_C, row_body, None)

        pltpu.sync_copy(
            rowbuf.at[pl.ds(0, ROWS_C * NCOLS)],
            out.at[pl.ds((r0 + c * ROWS_C) * NCOLS, ROWS_C * NCOLS)])
        return _

    lax.fori_loop(0, NCHUNK, chunk_body, None)


@jax.jit
def _run(idx_args, tbl_args, pt_args):
    mesh = plsc.VectorSubcoreMesh(core_axis_name="c", subcore_axis_name="s")
    scratch = [pltpu.VMEM((f["npad"],), jnp.int32) for f in EMB]
    scratch += [pltpu.VMEM((f["npad"], f["Dp"]), jnp.float32) for f in EMB]
    scratch += [pltpu.VMEM((BPW * f["S"] + 16,), jnp.float32) for f in PT]
    scratch += [pltpu.VMEM((ROWS_C * NCOLS + 16,), jnp.float32),
                pltpu.SemaphoreType.DMA]
    k = pl.kernel(
        _body,
        mesh=mesh,
        out_type=jax.ShapeDtypeStruct((B * NCOLS,), jnp.float32),
        scratch_types=scratch,
        compiler_params=pltpu.CompilerParams(use_tc_tiling_on_sc=False),
    )
    return k(*idx_args, *tbl_args, *pt_args).reshape(B, NCOLS)


def kernel(boil_time, mash_ph, sparge_temp, mash_step_type_inds,
           mash_step_times, mash_step_avg_temps, ferment_stage_times,
           ferment_stage_temps, grain_core_type_inds, grain_amts,
           adjunct_core_type_inds, adjunct_amts, hop_type_inds,
           hop_stage_type_inds, hop_times, hop_concentrations,
           misc_type_inds, misc_stage_inds, misc_times, misc_amts,
           mo_type_inds, mo_stage_inds, W_mash_step_type, W_grain_type,
           W_adjunct_type, W_hop_type, W_hop_stage_type, W_misc_type,
           W_misc_stage_type, W_mo_type, W_mo_stage_type):
    idx_inputs = [mash_step_type_inds, grain_core_type_inds,
                  adjunct_core_type_inds, hop_type_inds, hop_stage_type_inds,
                  misc_type_inds, misc_stage_inds, mo_type_inds,
                  mo_stage_inds]
    idx_args = [_arrange(a) for a in idx_inputs]
    tbls = [W_mash_step_type, W_grain_type, W_adjunct_type, W_hop_type,
            W_hop_stage_type, W_misc_type, W_misc_stage_type, W_mo_type,
            W_mo_stage_type]
    tbl_args = [
        jnp.pad(t, ((0, 0), (0, f["Dp"] - f["D"]))) if f["Dp"] != f["D"] else t
        for t, f in zip(tbls, EMB)]
    pt_args = [boil_time, mash_ph, sparge_temp,
               mash_step_times.reshape(-1), mash_step_avg_temps.reshape(-1),
               ferment_stage_times.reshape(-1), ferment_stage_temps.reshape(-1),
               grain_amts.reshape(-1), adjunct_amts.reshape(-1),
               hop_times.reshape(-1), hop_concentrations.reshape(-1),
               misc_times.reshape(-1), misc_amts.reshape(-1)]
    return _run(idx_args, tbl_args, pt_args)
